# baseline (speedup 1.0000x reference)
import functools

import jax
import jax.numpy as jnp
from jax import lax
from jax.experimental import pallas as pl
from jax.experimental.pallas import tpu as pltpu
from jax.experimental.pallas import tpu_sc as plsc

D = 128
L = 16
NCORE = 2
NTILE = 16
NRANGE = 6
RPC = NRANGE // NCORE
G = 128
SUB = 1024
CW = 16


def _ranged_rows(n_dst):
    return ((n_dst + NRANGE - 1) // NRANGE + 127) // 128 * 128


def _sc_segment_sum(y, src, dst, n_dst):
    e = src.shape[0]
    assert e % NTILE == 0
    r = _ranged_rows(n_dst)
    tr = (r + 64 + 63) // 64 * 64
    rpt = r // NTILE
    nzb = tr // 32
    nzit = (nzb + NTILE - 1) // NTILE
    c_per = e // NTILE
    nsub = c_per // SUB
    rem = c_per - nsub * SUB
    rem_blk = rem // G
    rem_tail = rem - rem_blk * G

    mesh = plsc.VectorSubcoreMesh(core_axis_name="c", subcore_axis_name="s")

    @functools.partial(
        pl.kernel,
        out_type=(
            jax.ShapeDtypeStruct((NRANGE * r, D), jnp.float32),
            jax.ShapeDtypeStruct((NRANGE * r, CW), jnp.float32),
        ),
        mesh=mesh,
        compiler_params=pltpu.CompilerParams(use_tc_tiling_on_sc=False),
        scratch_types=[
            pltpu.VMEM((SUB,), jnp.int32),
            pltpu.VMEM((SUB,), jnp.int32),
            pltpu.VMEM((G,), jnp.int32),
            pltpu.VMEM((G,), jnp.int32),
            pltpu.VMEM((G, D), jnp.float32),
            pltpu.VMEM((G, D), jnp.float32),
            pltpu.VMEM((G, CW), jnp.float32),
            pltpu.VMEM_SHARED((tr, D), jnp.float32),
            pltpu.VMEM_SHARED((tr, CW), jnp.float32),
            pltpu.SemaphoreType.DMA,
            pltpu.SemaphoreType.DMA,
            pltpu.SemaphoreType.DMA,
            pltpu.SemaphoreType.DMA,
        ],
    )
    def k(y_hbm, src_hbm, dst_hbm, acc_hbm, cnt_hbm,
          sub_s, sub_d, idx_a, idx_b, rows_a, rows_b, ones_v, table, cnts,
          semg_a, semg_b, semc_a, semc_b):
        t = lax.axis_index("s")
        core = lax.axis_index("c")
        zero16 = jnp.zeros((L,), jnp.float32)
        one16 = jnp.full((L,), 1.0, jnp.float32)
        base_e = t * c_per

        def init_ones(i, carry):
            ones_v[i, :] = one16
            return carry

        lax.fori_loop(0, G, init_ones, 0)

        for p in range(RPC):
            rblk = core * RPC + p
            lo = rblk * r

            def zrows(i, carry):
                for j in range(D // L):
                    rows_a[i, pl.ds(j * L, L)] = zero16
                return carry

            lax.fori_loop(0, G, zrows, 0)

            def ztbl(j, carry):
                blk = j * NTILE + t

                @pl.when(blk < nzb)
                def _():
                    pltpu.sync_copy(rows_a.at[pl.ds(0, 32), pl.ds(0, CW)],
                                    cnts.at[pl.ds(blk * 32, 32)])
                    pltpu.sync_copy(rows_a.at[pl.ds(0, 32)],
                                    table.at[pl.ds(blk * 32, 32)])

                return carry

            lax.fori_loop(0, nzit, ztbl, 0)
            plsc.subcore_barrier()

            def transform(base, ia):
                for j in range(G // L):
                    d = sub_d[pl.ds(base + j * L, L)]
                    m = (d >= lo) & (d < lo + r)
                    ia[pl.ds(j * L, L)] = jnp.where(m, d - lo, r)

            def pipeline(nb):
                bufs = [(idx_a, rows_a, semg_a, semc_a),
                        (idx_b, rows_b, semg_b, semc_b)]
                gd = {}
                cd = {}
                for bb in range(nb):
                    ia, ra, sg, sc = bufs[bb % 2]
                    if bb >= 2:
                        cd[bb - 2].wait()
                    transform(bb * G, ia)
                    gidx = sub_s.at[pl.ds(bb * G, G)]
                    gd[bb] = pltpu.async_copy(y_hbm.at[gidx], ra, sg)
                    if bb >= 1:
                        ip, rp, _, scp = bufs[(bb - 1) % 2]
                        gd[bb - 1].wait()
                        cd[bb - 1] = pltpu.async_copy(ones_v, cnts.at[ip],
                                                      scp, add=True)
                        pltpu.sync_copy(rp, table.at[ip], add=True)
                il, rl, _, scl = bufs[(nb - 1) % 2]
                gd[nb - 1].wait()
                cd[nb - 1] = pltpu.async_copy(ones_v, cnts.at[il],
                                              scl, add=True)
                pltpu.sync_copy(rl, table.at[il], add=True)
                if nb >= 2:
                    cd[nb - 2].wait()
                cd[nb - 1].wait()

            def sub_body(s, carry):
                off = base_e + s * SUB
                pltpu.sync_copy(src_hbm.at[pl.ds(off, SUB)], sub_s)
                pltpu.sync_copy(dst_hbm.at[pl.ds(off, SUB)], sub_d)
                pipeline(SUB // G)
                return carry

            lax.fori_loop(0, nsub, sub_body, 0)

            if rem:
                off = base_e + nsub * SUB
                pltpu.sync_copy(src_hbm.at[pl.ds(off, rem)],
                                sub_s.at[pl.ds(0, rem)])
                pltpu.sync_copy(dst_hbm.at[pl.ds(off, rem)],
                                sub_d.at[pl.ds(0, rem)])
                if rem_tail:
                    for o in range(rem, rem_blk * G + G, L):
                        sub_s[pl.ds(o, L)] = jnp.zeros((L,), jnp.int32)
                        sub_d[pl.ds(o, L)] = jnp.full((L,), -1, jnp.int32)
                pipeline(rem_blk + (1 if rem_tail else 0))

            plsc.subcore_barrier()

            off = t * rpt
            pltpu.sync_copy(table.at[pl.ds(off, rpt)],
                            acc_hbm.at[pl.ds(lo + off, rpt)])
            pltpu.sync_copy(cnts.at[pl.ds(off, rpt)],
                            cnt_hbm.at[pl.ds(lo + off, rpt)])
            plsc.subcore_barrier()

    return k(y, src, dst)


def _mm_kernel(x_ref, w_ref, o_ref):
    o_ref[...] = jnp.dot(x_ref[...], w_ref[...],
                         preferred_element_type=jnp.float32)


def _matmul(x, w):
    n = x.shape[0]
    bn = 1000
    assert n % bn == 0
    return pl.pallas_call(
        _mm_kernel,
        grid=(n // bn,),
        in_specs=[pl.BlockSpec((bn, D), lambda i: (i, 0)),
                  pl.BlockSpec((D, D), lambda i: (0, 0))],
        out_specs=pl.BlockSpec((bn, D), lambda i: (i, 0)),
        out_shape=jax.ShapeDtypeStruct((n, D), jnp.float32),
    )(x, w)


def _combine_kernel(acc_ref, cnt_ref, x_ref, w_ref, b_ref, o_ref):
    cnt = cnt_ref[...][:, 0:1]
    inv = 1.0 / jnp.maximum(cnt, 1.0)
    o_ref[...] = (acc_ref[...] * inv
                  + jnp.dot(x_ref[...], w_ref[...],
                            preferred_element_type=jnp.float32)
                  + b_ref[...])


def _combine(acc, cnt, x, w, b):
    n = x.shape[0]
    bn = 1000
    assert n % bn == 0
    return pl.pallas_call(
        _combine_kernel,
        grid=(n // bn,),
        in_specs=[pl.BlockSpec((bn, D), lambda i: (i, 0)),
                  pl.BlockSpec((bn, CW), lambda i: (i, 0)),
                  pl.BlockSpec((bn, D), lambda i: (i, 0)),
                  pl.BlockSpec((D, D), lambda i: (0, 0)),
                  pl.BlockSpec((1, D), lambda i: (0, 0))],
        out_specs=pl.BlockSpec((bn, D), lambda i: (i, 0)),
        out_shape=jax.ShapeDtypeStruct((n, D), jnp.float32),
    )(acc, cnt, x, w, b.reshape(1, D))


def kernel(x_user, x_item, edge_index_user_item, edge_index_item_user,
           Wl_u2i, Wr_u2i, b_u2i, Wl_i2u, Wr_i2u, b_i2u):
    n_user = x_user.shape[0]
    n_item = x_item.shape[0]
    src_ui = edge_index_user_item[0].astype(jnp.int32)
    dst_ui = edge_index_user_item[1].astype(jnp.int32)
    src_iu = edge_index_item_user[0].astype(jnp.int32)
    dst_iu = edge_index_item_user[1].astype(jnp.int32)

    y_u = _matmul(x_user, Wl_u2i)
    acc_i, cnt_i = _sc_segment_sum(y_u, src_ui, dst_ui, n_item)
    out_item = _combine(acc_i, cnt_i, x_item, Wr_u2i, b_u2i)

    y_i = _matmul(x_item, Wl_i2u)
    acc_u, cnt_u = _sc_segment_sum(y_i, src_iu, dst_iu, n_user)
    out_user = _combine(acc_u, cnt_u, x_user, Wr_i2u, b_i2u)

    return (out_user, out_item)


# speedup vs baseline: 1.7709x; 1.7709x over previous
import functools

import jax
import jax.numpy as jnp
from jax import lax
from jax.experimental import pallas as pl
from jax.experimental.pallas import tpu as pltpu
from jax.experimental.pallas import tpu_sc as plsc

D = 128
L = 16
NCORE = 2
NTILE = 16
NRANGE = 4
RPC = NRANGE // NCORE
G = 48
SUB = 768
CW = 16


def _ranged_rows(n_dst):
    return ((n_dst + NRANGE - 1) // NRANGE + 127) // 128 * 128


def _sc_segment_sum(y, src, dst, n_dst):
    e = src.shape[0]
    assert e % NTILE == 0
    r = _ranged_rows(n_dst)
    tr = (r + 64 + 63) // 64 * 64
    rpt = r // NTILE
    nzb = tr // 32
    nzit = (nzb + NTILE - 1) // NTILE
    c_per = e // NTILE
    nsub = c_per // SUB
    assert nsub % 2 == 0
    rem = c_per - nsub * SUB
    rem_blk = rem // G
    rem_tail = rem - rem_blk * G

    mesh = plsc.VectorSubcoreMesh(core_axis_name="c", subcore_axis_name="s")

    @functools.partial(
        pl.kernel,
        out_type=(
            jax.ShapeDtypeStruct((NRANGE * r, D), jnp.float32),
            jax.ShapeDtypeStruct((NRANGE * r, CW), jnp.float32),
        ),
        mesh=mesh,
        compiler_params=pltpu.CompilerParams(use_tc_tiling_on_sc=False),
        scratch_types=[
            pltpu.VMEM((SUB,), jnp.int32),
            pltpu.VMEM((SUB,), jnp.int32),
            pltpu.VMEM((SUB,), jnp.int32),
            pltpu.VMEM((SUB,), jnp.int32),
            pltpu.VMEM((G,), jnp.int32),
            pltpu.VMEM((G,), jnp.int32),
            pltpu.VMEM((G, D), jnp.float32),
            pltpu.VMEM((G, D), jnp.float32),
            pltpu.VMEM((G, CW), jnp.float32),
            pltpu.VMEM_SHARED((tr, D), jnp.float32),
            pltpu.VMEM_SHARED((tr, CW), jnp.float32),
            pltpu.SemaphoreType.DMA,
            pltpu.SemaphoreType.DMA,
            pltpu.SemaphoreType.DMA,
            pltpu.SemaphoreType.DMA,
            pltpu.SemaphoreType.DMA,
            pltpu.SemaphoreType.DMA,
        ],
    )
    def k(y_hbm, src_hbm, dst_hbm, acc_hbm, cnt_hbm,
          sub_sa, sub_da, sub_sb, sub_db, idx_a, idx_b, rows_a, rows_b,
          ones_v, table, cnts, semg_a, semg_b, semc_a, semc_b,
          semi_a, semi_b):
        t = lax.axis_index("s")
        core = lax.axis_index("c")
        zero16 = jnp.zeros((L,), jnp.float32)
        one16 = jnp.full((L,), 1.0, jnp.float32)
        base_e = t * c_per

        def init_ones(i, carry):
            ones_v[i, :] = one16
            return carry

        lax.fori_loop(0, G, init_ones, 0)

        for p in range(RPC):
            rblk = core * RPC + p
            lo = rblk * r

            def zrows(i, carry):
                for j in range(D // L):
                    rows_a[i, pl.ds(j * L, L)] = zero16
                return carry

            lax.fori_loop(0, G, zrows, 0)

            def ztbl(j, carry):
                blk = j * NTILE + t

                @pl.when(blk < nzb)
                def _():
                    pltpu.sync_copy(rows_a.at[pl.ds(0, 32), pl.ds(0, CW)],
                                    cnts.at[pl.ds(blk * 32, 32)])
                    pltpu.sync_copy(rows_a.at[pl.ds(0, 32)],
                                    table.at[pl.ds(blk * 32, 32)])

                return carry

            lax.fori_loop(0, nzit, ztbl, 0)
            plsc.subcore_barrier()

            def transform(sub_d, base, ia):
                for j in range(G // L):
                    d = sub_d[pl.ds(base + j * L, L)]
                    m = (d >= lo) & (d < lo + r)
                    ia[pl.ds(j * L, L)] = jnp.where(m, d - lo, r)

            def pipeline(sub_s, sub_d, nb):
                bufs = [(idx_a, rows_a, semg_a, semc_a),
                        (idx_b, rows_b, semg_b, semc_b)]
                gd = {}
                cd = {}
                for bb in range(nb):
                    ia, ra, sg, sc = bufs[bb % 2]
                    if bb >= 2:
                        cd[bb - 2].wait()
                    transform(sub_d, bb * G, ia)
                    gidx = sub_s.at[pl.ds(bb * G, G)]
                    gd[bb] = pltpu.async_copy(y_hbm.at[gidx], ra, sg)
                    if bb >= 1:
                        ip, rp, _, scp = bufs[(bb - 1) % 2]
                        gd[bb - 1].wait()
                        cd[bb - 1] = pltpu.async_copy(ones_v, cnts.at[ip],
                                                      scp, add=True)
                        pltpu.sync_copy(rp, table.at[ip], add=True)
                il, rl, _, scl = bufs[(nb - 1) % 2]
                gd[nb - 1].wait()
                cd[nb - 1] = pltpu.async_copy(ones_v, cnts.at[il],
                                              scl, add=True)
                pltpu.sync_copy(rl, table.at[il], add=True)
                if nb >= 2:
                    cd[nb - 2].wait()
                cd[nb - 1].wait()

            def sub_body(s2, carry):
                offa = base_e + (2 * s2) * SUB
                offb = offa + SUB
                la1 = pltpu.async_copy(src_hbm.at[pl.ds(offa, SUB)],
                                       sub_sa, semi_a)
                la2 = pltpu.async_copy(dst_hbm.at[pl.ds(offa, SUB)],
                                       sub_da, semi_a)
                lb1 = pltpu.async_copy(src_hbm.at[pl.ds(offb, SUB)],
                                       sub_sb, semi_b)
                lb2 = pltpu.async_copy(dst_hbm.at[pl.ds(offb, SUB)],
                                       sub_db, semi_b)
                la1.wait()
                la2.wait()
                pipeline(sub_sa, sub_da, SUB // G)
                lb1.wait()
                lb2.wait()
                pipeline(sub_sb, sub_db, SUB // G)
                return carry

            lax.fori_loop(0, nsub // 2, sub_body, 0)

            if rem:
                off = base_e + nsub * SUB
                pltpu.sync_copy(src_hbm.at[pl.ds(off, rem)],
                                sub_sa.at[pl.ds(0, rem)])
                pltpu.sync_copy(dst_hbm.at[pl.ds(off, rem)],
                                sub_da.at[pl.ds(0, rem)])
                if rem_tail:
                    for o in range(rem, rem_blk * G + G, L):
                        sub_sa[pl.ds(o, L)] = jnp.zeros((L,), jnp.int32)
                        sub_da[pl.ds(o, L)] = jnp.full((L,), -1, jnp.int32)
                pipeline(sub_sa, sub_da, rem_blk + (1 if rem_tail else 0))

            plsc.subcore_barrier()

            off = t * rpt
            pltpu.sync_copy(table.at[pl.ds(off, rpt)],
                            acc_hbm.at[pl.ds(lo + off, rpt)])
            pltpu.sync_copy(cnts.at[pl.ds(off, rpt)],
                            cnt_hbm.at[pl.ds(lo + off, rpt)])
            plsc.subcore_barrier()

    return k(y, src, dst)


def _mm_kernel(x_ref, w_ref, o_ref):
    o_ref[...] = jnp.dot(x_ref[...], w_ref[...],
                         preferred_element_type=jnp.float32)


def _matmul(x, w):
    n = x.shape[0]
    bn = 1000
    assert n % bn == 0
    return pl.pallas_call(
        _mm_kernel,
        grid=(n // bn,),
        in_specs=[pl.BlockSpec((bn, D), lambda i: (i, 0)),
                  pl.BlockSpec((D, D), lambda i: (0, 0))],
        out_specs=pl.BlockSpec((bn, D), lambda i: (i, 0)),
        out_shape=jax.ShapeDtypeStruct((n, D), jnp.float32),
    )(x, w)


def _combine_kernel(acc_ref, cnt_ref, x_ref, w_ref, b_ref, o_ref):
    cnt = cnt_ref[...][:, 0:1]
    inv = 1.0 / jnp.maximum(cnt, 1.0)
    o_ref[...] = (acc_ref[...] * inv
                  + jnp.dot(x_ref[...], w_ref[...],
                            preferred_element_type=jnp.float32)
                  + b_ref[...])


def _combine(acc, cnt, x, w, b):
    n = x.shape[0]
    bn = 1000
    assert n % bn == 0
    return pl.pallas_call(
        _combine_kernel,
        grid=(n // bn,),
        in_specs=[pl.BlockSpec((bn, D), lambda i: (i, 0)),
                  pl.BlockSpec((bn, CW), lambda i: (i, 0)),
                  pl.BlockSpec((bn, D), lambda i: (i, 0)),
                  pl.BlockSpec((D, D), lambda i: (0, 0)),
                  pl.BlockSpec((1, D), lambda i: (0, 0))],
        out_specs=pl.BlockSpec((bn, D), lambda i: (i, 0)),
        out_shape=jax.ShapeDtypeStruct((n, D), jnp.float32),
    )(acc, cnt, x, w, b.reshape(1, D))


def kernel(x_user, x_item, edge_index_user_item, edge_index_item_user,
           Wl_u2i, Wr_u2i, b_u2i, Wl_i2u, Wr_i2u, b_i2u):
    n_user = x_user.shape[0]
    n_item = x_item.shape[0]
    src_ui = edge_index_user_item[0].astype(jnp.int32)
    dst_ui = edge_index_user_item[1].astype(jnp.int32)
    src_iu = edge_index_item_user[0].astype(jnp.int32)
    dst_iu = edge_index_item_user[1].astype(jnp.int32)

    y_u = _matmul(x_user, Wl_u2i)
    acc_i, cnt_i = _sc_segment_sum(y_u, src_ui, dst_ui, n_item)
    out_item = _combine(acc_i, cnt_i, x_item, Wr_u2i, b_u2i)

    y_i = _matmul(x_item, Wl_i2u)
    acc_u, cnt_u = _sc_segment_sum(y_i, src_iu, dst_iu, n_user)
    out_user = _combine(acc_u, cnt_u, x_user, Wr_i2u, b_i2u)

    return (out_user, out_item)
